# baseline (speedup 1.0000x reference)
import functools

import jax
import jax.numpy as jnp
from jax import lax
from jax.experimental import pallas as pl
from jax.experimental.pallas import tpu as pltpu
from jax.experimental.pallas import tpu_sc as plsc

N_ATOMS = 3200000
N_PROPS = 14
ROW_PAD = 16
N_ROWS = 86
BLK = 128
NBLK = N_ATOMS // BLK
CB = 20
CHUNK = CB * BLK
NCHUNK = NBLK // CB
L = 16


def _make_kernel(nc, ns):
    nw = nc * ns
    per_w = -(-NCHUNK // nw)
    if per_w % 2:
        per_w += 1
    half = per_w // 2
    mesh = plsc.VectorSubcoreMesh(core_axis_name="c", subcore_axis_name="s")

    @functools.partial(
        pl.kernel,
        mesh=mesh,
        compiler_params=pltpu.CompilerParams(needs_layout_passes=False,
                                             use_tc_tiling_on_sc=False),
        out_type=(
            jax.ShapeDtypeStruct((N_ATOMS,), jnp.int32),
            jax.ShapeDtypeStruct((N_ATOMS,), jnp.int32),
            jax.ShapeDtypeStruct((2, NBLK, 8, BLK), jnp.float32),
        ),
        scratch_types=[
            pltpu.VMEM((2, CB, BLK), jnp.int32),
            pltpu.VMEM((2, CHUNK), jnp.int32),
            pltpu.VMEM((2, CHUNK), jnp.int32),
            pltpu.VMEM((2, CB, 8, BLK), jnp.float32),
            pltpu.VMEM((2, CB, 8, BLK), jnp.float32),
            pltpu.VMEM((N_ROWS * ROW_PAD,), jnp.int32),
            pltpu.SemaphoreType.DMA,
            pltpu.SemaphoreType.DMA,
            pltpu.SemaphoreType.DMA,
            pltpu.SemaphoreType.DMA,
        ],
    )
    def phys_ref_sc(z2_hbm, tab_hbm, period_out, group_out, props_out,
                    z_v, per_v, grp_v, p0_v, p1_v, tab_v,
                    zsem0, zsem1, osem0, osem1):
        wid = lax.axis_index("s") * nc + lax.axis_index("c")
        start = wid * per_w

        pltpu.sync_copy(tab_hbm, tab_v)

        zsem = (zsem0, zsem1)
        osem = (osem0, osem1)

        def geff(t):
            g = start + t
            return jnp.where(g < NCHUNK, g, start)

        def z_copy(p, g):
            return pltpu.make_async_copy(
                z2_hbm.at[pl.ds(g * CB, CB)], z_v.at[p], zsem[p])

        def out_copies(p, g):
            return (
                pltpu.make_async_copy(
                    per_v.at[p], period_out.at[pl.ds(g * CHUNK, CHUNK)],
                    osem[p]),
                pltpu.make_async_copy(
                    grp_v.at[p], group_out.at[pl.ds(g * CHUNK, CHUNK)],
                    osem[p]),
                pltpu.make_async_copy(
                    p0_v.at[p], props_out.at[0, pl.ds(g * CB, CB)], osem[p]),
                pltpu.make_async_copy(
                    p1_v.at[p], props_out.at[1, pl.ds(g * CB, CB)], osem[p]),
            )

        def compute(p):
            @plsc.parallel_loop(0, CHUNK // L, unroll=8)
            def body(t):
                blk = t // (BLK // L)
                gg = t % (BLK // L)
                za = z_v[p, blk, pl.ds(gg * L, L)]
                base = za * ROW_PAD
                for j in range(N_PROPS):
                    v = plsc.load_gather(tab_v, [base + j])
                    vf = plsc.bitcast(v, jnp.float32)
                    if j < 8:
                        p0_v[p, blk, j, pl.ds(gg * L, L)] = vf
                    else:
                        p1_v[p, blk, j - 8, pl.ds(gg * L, L)] = vf
                per_v[p, pl.ds(t * L, L)] = plsc.load_gather(
                    tab_v, [base + N_PROPS])
                grp_v[p, pl.ds(t * L, L)] = plsc.load_gather(
                    tab_v, [base + N_PROPS + 1])

        def finish(p, g, k):
            z_copy(p, g).wait()
            @pl.when(k > 0)
            def _():
                for cp in out_copies(p, g):
                    cp.wait()
            compute(p)
            for cp in out_copies(p, g):
                cp.start()

        z_copy(0, geff(0)).start()

        def body2(k, carry):
            g0 = geff(2 * k)
            g1 = geff(2 * k + 1)
            g2 = geff(2 * k + 2)
            z_copy(1, g1).start()
            finish(0, g0, k)
            @pl.when(k < half - 1)
            def _():
                z_copy(0, g2).start()
            finish(1, g1, k)
            return carry

        lax.fori_loop(0, half, body2, 0)

        for cp in out_copies(0, geff(per_w - 2)):
            cp.wait()
        for cp in out_copies(1, geff(per_w - 1)):
            cp.wait()

    return phys_ref_sc


def kernel(z, period_mapping, group_mapping, properties_mapping):
    info = plsc.get_sparse_core_info()
    f = _make_kernel(info.num_cores, info.num_subcores)
    tab = jnp.concatenate([
        jax.lax.bitcast_convert_type(properties_mapping, jnp.int32),
        period_mapping.astype(jnp.int32)[:, None],
        group_mapping.astype(jnp.int32)[:, None],
    ], axis=1).reshape(N_ROWS * ROW_PAD)
    z2 = z.reshape(NBLK, BLK)
    period, group, planes = f(z2, tab)
    props = planes.transpose(1, 3, 0, 2).reshape(N_ATOMS, 16)[:, :N_PROPS]
    return (period, group, props)


# speedup vs baseline: 1.1787x; 1.1787x over previous
import functools

import jax
import jax.numpy as jnp
from jax import lax
from jax.experimental import pallas as pl
from jax.experimental.pallas import tpu as pltpu
from jax.experimental.pallas import tpu_sc as plsc

N_ATOMS = 3200000
N_PROPS = 14
ROW_PAD = 16
N_ROWS = 86
BLK = 128
NBLK = N_ATOMS // BLK
CB = 20
CHUNK = CB * BLK
NCHUNK = NBLK // CB
L = 16


def _make_kernel(nc, ns):
    nw = nc * ns
    per_w = -(-NCHUNK // nw)
    if per_w % 2:
        per_w += 1
    half = per_w // 2
    mesh = plsc.VectorSubcoreMesh(core_axis_name="c", subcore_axis_name="s")

    @functools.partial(
        pl.kernel,
        mesh=mesh,
        compiler_params=pltpu.CompilerParams(needs_layout_passes=False,
                                             use_tc_tiling_on_sc=False),
        out_type=(
            jax.ShapeDtypeStruct((N_ATOMS,), jnp.int32),
            jax.ShapeDtypeStruct((N_ATOMS,), jnp.int32),
            jax.ShapeDtypeStruct((2, NBLK, 8, BLK), jnp.float32),
        ),
        scratch_types=[
            pltpu.VMEM((2, CB, BLK), jnp.int32),
            pltpu.VMEM((2, CHUNK), jnp.int32),
            pltpu.VMEM((2, CHUNK), jnp.int32),
            pltpu.VMEM((2, CB, 8, BLK), jnp.float32),
            pltpu.VMEM((2, CB, 8, BLK), jnp.float32),
            pltpu.VMEM((N_ROWS * ROW_PAD,), jnp.int32),
            pltpu.SemaphoreType.DMA,
            pltpu.SemaphoreType.DMA,
            pltpu.SemaphoreType.DMA,
            pltpu.SemaphoreType.DMA,
        ],
    )
    def phys_ref_sc(z2_hbm, tab_hbm, period_out, group_out, props_out,
                    z_v, per_v, grp_v, p0_v, p1_v, tab_v,
                    zsem0, zsem1, osem0, osem1):
        wid = lax.axis_index("s") * nc + lax.axis_index("c")
        start = wid * per_w

        pltpu.sync_copy(tab_hbm, tab_v)

        zsem = (zsem0, zsem1)
        osem = (osem0, osem1)

        def geff(t):
            g = start + t
            return jnp.where(g < NCHUNK, g, start)

        def z_copy(p, g):
            return pltpu.make_async_copy(
                z2_hbm.at[pl.ds(g * CB, CB)], z_v.at[p], zsem[p])

        def out_copies(p, g):
            return (
                pltpu.make_async_copy(
                    per_v.at[p], period_out.at[pl.ds(g * CHUNK, CHUNK)],
                    osem[p]),
                pltpu.make_async_copy(
                    grp_v.at[p], group_out.at[pl.ds(g * CHUNK, CHUNK)],
                    osem[p]),
                pltpu.make_async_copy(
                    p0_v.at[p], props_out.at[0, pl.ds(g * CB, CB)], osem[p]),
                pltpu.make_async_copy(
                    p1_v.at[p], props_out.at[1, pl.ds(g * CB, CB)], osem[p]),
            )

        def compute(p):
            @plsc.parallel_loop(0, CHUNK // L, unroll=4)
            def body(t):
                blk = t // (BLK // L)
                gg = t % (BLK // L)
                za = z_v[p, blk, pl.ds(gg * L, L)]
                base = za * ROW_PAD
                for j in range(N_PROPS):
                    v = plsc.load_gather(tab_v, [base + j])
                    vf = plsc.bitcast(v, jnp.float32)
                    if j < 8:
                        p0_v[p, blk, j, pl.ds(gg * L, L)] = vf
                    else:
                        p1_v[p, blk, j - 8, pl.ds(gg * L, L)] = vf
                per_v[p, pl.ds(t * L, L)] = plsc.load_gather(
                    tab_v, [base + N_PROPS])
                grp_v[p, pl.ds(t * L, L)] = plsc.load_gather(
                    tab_v, [base + N_PROPS + 1])

        def finish(p, g, k):
            z_copy(p, g).wait()
            @pl.when(k > 0)
            def _():
                for cp in out_copies(p, g):
                    cp.wait()
            compute(p)
            for cp in out_copies(p, g):
                cp.start()

        z_copy(0, geff(0)).start()

        def body2(k, carry):
            g0 = geff(2 * k)
            g1 = geff(2 * k + 1)
            g2 = geff(2 * k + 2)
            z_copy(1, g1).start()
            finish(0, g0, k)
            @pl.when(k < half - 1)
            def _():
                z_copy(0, g2).start()
            finish(1, g1, k)
            return carry

        lax.fori_loop(0, half, body2, 0)

        for cp in out_copies(0, geff(per_w - 2)):
            cp.wait()
        for cp in out_copies(1, geff(per_w - 1)):
            cp.wait()

    return phys_ref_sc


def kernel(z, period_mapping, group_mapping, properties_mapping):
    info = plsc.get_sparse_core_info()
    f = _make_kernel(info.num_cores, info.num_subcores)
    tab = jnp.concatenate([
        jax.lax.bitcast_convert_type(properties_mapping, jnp.int32),
        period_mapping.astype(jnp.int32)[:, None],
        group_mapping.astype(jnp.int32)[:, None],
    ], axis=1).reshape(N_ROWS * ROW_PAD)
    z2 = z.reshape(NBLK, BLK)
    period, group, planes = f(z2, tab)
    props = planes.transpose(1, 3, 0, 2).reshape(N_ATOMS, 16)[:, :N_PROPS]
    return (period, group, props)


# speedup vs baseline: 1.5178x; 1.2876x over previous
import functools

import jax
import jax.numpy as jnp
from jax import lax
from jax.experimental import pallas as pl
from jax.experimental.pallas import tpu as pltpu
from jax.experimental.pallas import tpu_sc as plsc

N_ATOMS = 3200000
N_PROPS = 14
ROW_PAD = 16
N_ROWS = 86
BLK = 128
NBLK = N_ATOMS // BLK
CB = 20
CHUNK = CB * BLK
NCHUNK = NBLK // CB
L = 16


def _make_kernel(nc, ns):
    nw = nc * ns
    per_w = -(-NCHUNK // nw)
    if per_w % 2:
        per_w += 1
    half = per_w // 2
    mesh = plsc.VectorSubcoreMesh(core_axis_name="c", subcore_axis_name="s")

    @functools.partial(
        pl.kernel,
        mesh=mesh,
        compiler_params=pltpu.CompilerParams(needs_layout_passes=False,
                                             use_tc_tiling_on_sc=False),
        out_type=(
            jax.ShapeDtypeStruct((N_ATOMS,), jnp.int32),
            jax.ShapeDtypeStruct((N_ATOMS,), jnp.int32),
            jax.ShapeDtypeStruct((2, NBLK, 8, BLK), jnp.float32),
        ),
        scratch_types=[
            pltpu.VMEM((2, CB, BLK), jnp.int32),
            pltpu.VMEM((2, CHUNK), jnp.int32),
            pltpu.VMEM((2, CHUNK), jnp.int32),
            pltpu.VMEM((2, CB, 8, BLK), jnp.float32),
            pltpu.VMEM((2, CB, 8, BLK), jnp.float32),
            pltpu.VMEM((N_ROWS * ROW_PAD,), jnp.int32),
            pltpu.SemaphoreType.DMA,
            pltpu.SemaphoreType.DMA,
            pltpu.SemaphoreType.DMA,
            pltpu.SemaphoreType.DMA,
        ],
    )
    def phys_ref_sc(z2_hbm, tab_hbm, period_out, group_out, props_out,
                    z_v, per_v, grp_v, p0_v, p1_v, tab_v,
                    zsem0, zsem1, osem0, osem1):
        wid = lax.axis_index("s") * nc + lax.axis_index("c")
        start = wid * per_w

        pltpu.sync_copy(tab_hbm, tab_v)

        zsem = (zsem0, zsem1)
        osem = (osem0, osem1)

        def geff(t):
            g = start + t
            return jnp.where(g < NCHUNK, g, start)

        def z_copy(p, g):
            return pltpu.make_async_copy(
                z2_hbm.at[pl.ds(g * CB, CB)], z_v.at[p], zsem[p])

        def out_copies(p, g):
            return (
                pltpu.make_async_copy(
                    per_v.at[p], period_out.at[pl.ds(g * CHUNK, CHUNK)],
                    osem[p]),
                pltpu.make_async_copy(
                    grp_v.at[p], group_out.at[pl.ds(g * CHUNK, CHUNK)],
                    osem[p]),
                pltpu.make_async_copy(
                    p0_v.at[p], props_out.at[0, pl.ds(g * CB, CB)], osem[p]),
                pltpu.make_async_copy(
                    p1_v.at[p], props_out.at[1, pl.ds(g * CB, CB)], osem[p]),
            )

        def compute(p):
            @plsc.parallel_loop(0, CHUNK // L, unroll=2)
            def body(t):
                blk = t // (BLK // L)
                gg = t % (BLK // L)
                za = z_v[p, blk, pl.ds(gg * L, L)]
                base = za * ROW_PAD
                for j in range(N_PROPS):
                    v = plsc.load_gather(tab_v, [base + j])
                    vf = plsc.bitcast(v, jnp.float32)
                    if j < 8:
                        p0_v[p, blk, j, pl.ds(gg * L, L)] = vf
                    else:
                        p1_v[p, blk, j - 8, pl.ds(gg * L, L)] = vf
                per_v[p, pl.ds(t * L, L)] = plsc.load_gather(
                    tab_v, [base + N_PROPS])
                grp_v[p, pl.ds(t * L, L)] = plsc.load_gather(
                    tab_v, [base + N_PROPS + 1])

        def finish(p, g, k):
            z_copy(p, g).wait()
            @pl.when(k > 0)
            def _():
                for cp in out_copies(p, g):
                    cp.wait()
            compute(p)
            for cp in out_copies(p, g):
                cp.start()

        z_copy(0, geff(0)).start()

        def body2(k, carry):
            g0 = geff(2 * k)
            g1 = geff(2 * k + 1)
            g2 = geff(2 * k + 2)
            z_copy(1, g1).start()
            finish(0, g0, k)
            @pl.when(k < half - 1)
            def _():
                z_copy(0, g2).start()
            finish(1, g1, k)
            return carry

        lax.fori_loop(0, half, body2, 0)

        for cp in out_copies(0, geff(per_w - 2)):
            cp.wait()
        for cp in out_copies(1, geff(per_w - 1)):
            cp.wait()

    return phys_ref_sc


def kernel(z, period_mapping, group_mapping, properties_mapping):
    info = plsc.get_sparse_core_info()
    f = _make_kernel(info.num_cores, info.num_subcores)
    tab = jnp.concatenate([
        jax.lax.bitcast_convert_type(properties_mapping, jnp.int32),
        period_mapping.astype(jnp.int32)[:, None],
        group_mapping.astype(jnp.int32)[:, None],
    ], axis=1).reshape(N_ROWS * ROW_PAD)
    z2 = z.reshape(NBLK, BLK)
    period, group, planes = f(z2, tab)
    props = planes.transpose(1, 3, 0, 2).reshape(N_ATOMS, 16)[:, :N_PROPS]
    return (period, group, props)


# speedup vs baseline: 1.5276x; 1.0064x over previous
import functools

import jax
import jax.numpy as jnp
from jax import lax
from jax.experimental import pallas as pl
from jax.experimental.pallas import tpu as pltpu
from jax.experimental.pallas import tpu_sc as plsc

N_ATOMS = 3200000
N_PROPS = 14
ROW_PAD = 16
N_ROWS = 86
BLK = 128
NBLK = N_ATOMS // BLK
CB = 20
CHUNK = CB * BLK
NCHUNK = NBLK // CB
L = 16


def _make_kernel(nc, ns):
    nw = nc * ns
    per_w = -(-NCHUNK // nw)
    if per_w % 2:
        per_w += 1
    half = per_w // 2
    mesh = plsc.VectorSubcoreMesh(core_axis_name="c", subcore_axis_name="s")

    @functools.partial(
        pl.kernel,
        mesh=mesh,
        compiler_params=pltpu.CompilerParams(needs_layout_passes=False,
                                             use_tc_tiling_on_sc=False),
        out_type=(
            jax.ShapeDtypeStruct((N_ATOMS,), jnp.int32),
            jax.ShapeDtypeStruct((N_ATOMS,), jnp.int32),
            jax.ShapeDtypeStruct((2, NBLK, 8, BLK), jnp.float32),
        ),
        scratch_types=[
            pltpu.VMEM((2, CB, BLK), jnp.int32),
            pltpu.VMEM((2, CHUNK), jnp.int32),
            pltpu.VMEM((2, CHUNK), jnp.int32),
            pltpu.VMEM((2, CB, 8, BLK), jnp.float32),
            pltpu.VMEM((2, CB, 8, BLK), jnp.float32),
            pltpu.VMEM((N_ROWS * ROW_PAD,), jnp.int32),
            pltpu.SemaphoreType.DMA,
            pltpu.SemaphoreType.DMA,
            pltpu.SemaphoreType.DMA,
            pltpu.SemaphoreType.DMA,
        ],
    )
    def phys_ref_sc(z2_hbm, tab_hbm, period_out, group_out, props_out,
                    z_v, per_v, grp_v, p0_v, p1_v, tab_v,
                    zsem0, zsem1, osem0, osem1):
        wid = lax.axis_index("s") * nc + lax.axis_index("c")
        start = wid * per_w

        pltpu.sync_copy(tab_hbm, tab_v)

        zsem = (zsem0, zsem1)
        osem = (osem0, osem1)

        def geff(t):
            g = start + t
            return jnp.where(g < NCHUNK, g, start)

        def z_copy(p, g):
            return pltpu.make_async_copy(
                z2_hbm.at[pl.ds(g * CB, CB)], z_v.at[p], zsem[p])

        def out_copies(p, g):
            return (
                pltpu.make_async_copy(
                    per_v.at[p], period_out.at[pl.ds(g * CHUNK, CHUNK)],
                    osem[p]),
                pltpu.make_async_copy(
                    grp_v.at[p], group_out.at[pl.ds(g * CHUNK, CHUNK)],
                    osem[p]),
                pltpu.make_async_copy(
                    p0_v.at[p], props_out.at[0, pl.ds(g * CB, CB)], osem[p]),
                pltpu.make_async_copy(
                    p1_v.at[p], props_out.at[1, pl.ds(g * CB, CB)], osem[p]),
            )

        def compute(p):
            @plsc.parallel_loop(0, CHUNK // L, unroll=1)
            def body(t):
                blk = t // (BLK // L)
                gg = t % (BLK // L)
                za = z_v[p, blk, pl.ds(gg * L, L)]
                base = za * ROW_PAD
                for j in range(N_PROPS):
                    v = plsc.load_gather(tab_v, [base + j])
                    vf = plsc.bitcast(v, jnp.float32)
                    if j < 8:
                        p0_v[p, blk, j, pl.ds(gg * L, L)] = vf
                    else:
                        p1_v[p, blk, j - 8, pl.ds(gg * L, L)] = vf
                per_v[p, pl.ds(t * L, L)] = plsc.load_gather(
                    tab_v, [base + N_PROPS])
                grp_v[p, pl.ds(t * L, L)] = plsc.load_gather(
                    tab_v, [base + N_PROPS + 1])

        def finish(p, g, k):
            z_copy(p, g).wait()
            @pl.when(k > 0)
            def _():
                for cp in out_copies(p, g):
                    cp.wait()
            compute(p)
            for cp in out_copies(p, g):
                cp.start()

        z_copy(0, geff(0)).start()

        def body2(k, carry):
            g0 = geff(2 * k)
            g1 = geff(2 * k + 1)
            g2 = geff(2 * k + 2)
            z_copy(1, g1).start()
            finish(0, g0, k)
            @pl.when(k < half - 1)
            def _():
                z_copy(0, g2).start()
            finish(1, g1, k)
            return carry

        lax.fori_loop(0, half, body2, 0)

        for cp in out_copies(0, geff(per_w - 2)):
            cp.wait()
        for cp in out_copies(1, geff(per_w - 1)):
            cp.wait()

    return phys_ref_sc


def kernel(z, period_mapping, group_mapping, properties_mapping):
    info = plsc.get_sparse_core_info()
    f = _make_kernel(info.num_cores, info.num_subcores)
    tab = jnp.concatenate([
        jax.lax.bitcast_convert_type(properties_mapping, jnp.int32),
        period_mapping.astype(jnp.int32)[:, None],
        group_mapping.astype(jnp.int32)[:, None],
    ], axis=1).reshape(N_ROWS * ROW_PAD)
    z2 = z.reshape(NBLK, BLK)
    period, group, planes = f(z2, tab)
    props = planes.transpose(1, 3, 0, 2).reshape(N_ATOMS, 16)[:, :N_PROPS]
    return (period, group, props)
